# Initial kernel scaffold; baseline (speedup 1.0000x reference)
import jax
import jax.numpy as jnp
from jax.experimental import pallas as pl


def kernel(points, point_features, query_points, W_skip, b_skip, W1, b1, W2, b2, W3, b3, Wo1, bo1, Wo2, bo2):
    raise NotImplementedError("write your pallas kernel here")


# speedup vs baseline: 12.4021x; 12.4021x over previous
import jax
import jax.numpy as jnp
from jax import lax
from jax.experimental import pallas as pl
from jax.experimental.pallas import tpu as pltpu
from jax.experimental.pallas import tpu_sc as plsc

_B, _N, _K = 4, 2048, 16
_CIN = 128
_CP = 256
_TNQ = 256
_TN = 256
_GW = 128


def _knn_body(q_ref, p_ref, o_ref):
    q = q_ref[0]
    p = p_ref[0]
    q2 = jnp.sum(q * q, axis=1, keepdims=True)
    p2 = jnp.sum(p * p, axis=0, keepdims=True)
    qp = jnp.dot(q, p, preferred_element_type=jnp.float32)
    d = q2 + p2 - 2.0 * qp
    iota = lax.broadcasted_iota(jnp.int32, d.shape, 1)
    for t in range(_K):
        m = jnp.min(d, axis=1, keepdims=True)
        am = jnp.min(jnp.where(d == m, iota, _N), axis=1, keepdims=True)
        o_ref[0, :, t] = am[:, 0]
        d = jnp.where(iota == am, jnp.float32(jnp.inf), d)


def _knn(q8, p8t):
    return pl.pallas_call(
        _knn_body,
        grid=(_B, _N // _TNQ),
        in_specs=[
            pl.BlockSpec((1, _TNQ, 8), lambda b, i: (b, i, 0)),
            pl.BlockSpec((1, 8, _N), lambda b, i: (b, 0, 0)),
        ],
        out_specs=pl.BlockSpec((1, _TNQ, _K), lambda b, i: (b, i, 0)),
        out_shape=jax.ShapeDtypeStruct((_B, _N, _K), jnp.int32),
    )(q8, p8t)


def _sc_gather(table, flat_idx):
    num = flat_idx.shape[1]

    @pl.kernel(
        out_type=jax.ShapeDtypeStruct((num, _CP), jnp.float32),
        mesh=plsc.VectorSubcoreMesh(core_axis_name="c", subcore_axis_name="s"),
    )
    def gather_kernel(t_hbm, i_hbm, o_hbm):
        def body(i_vmem, o_vmem):
            pltpu.sync_copy(t_hbm.at[i_vmem.at[0]], o_vmem)

        pltpu.emit_pipeline(
            body,
            grid=(num // _GW,),
            in_specs=[pl.BlockSpec((1, _GW), index_map=lambda i: (0, i))],
            out_specs=[pl.BlockSpec((_GW, _CP), index_map=lambda i: (i, 0))],
            core_axis_name=("c", "s"),
            dimension_semantics=(pltpu.PARALLEL,),
        )(i_hbm, o_hbm)

    return gather_kernel(table, flat_idx)


def _mlp_body(g_ref, pt_ref,
              w1f_ref, w1p_ref, w1q_ref, b1_ref,
              w2_ref, b2_ref, w3_ref, b3_ref,
              wsf_ref, wsp_ref, wsq_ref, bs_ref,
              wo1_ref, bo1_ref, wo2_ref, bo2_ref, o_ref):
    g = g_ref[0]
    gf = g[:, :, 0:_CIN]
    gp = g[:, :, _CIN:_CIN + 8]
    pt = pt_ref[0]

    gff = gf.reshape(_TN * _K, _CIN)
    gpf = gp.reshape(_TN * _K, 8)

    h = (jnp.dot(gff, w1f_ref[...], preferred_element_type=jnp.float32)
         + jnp.dot(gpf, w1p_ref[...], preferred_element_type=jnp.float32))
    h = h.reshape(_TN, _K, 128)
    h = h - jnp.dot(pt, w1q_ref[...], preferred_element_type=jnp.float32)[:, None, :]
    h = jnp.maximum(h + b1_ref[...], 0.0)
    h = h.reshape(_TN * _K, 128)
    h = jnp.maximum(jnp.dot(h, w2_ref[...], preferred_element_type=jnp.float32)
                    + b2_ref[...], 0.0)
    h = jnp.maximum(jnp.dot(h, w3_ref[...], preferred_element_type=jnp.float32)
                    + b3_ref[...], 0.0)
    h = h.reshape(_TN, _K, 256)

    gfm = jnp.max(gf, axis=1)
    gpm = jnp.max(gp, axis=1)
    sk = (jnp.dot(gfm, wsf_ref[...], preferred_element_type=jnp.float32)
          + jnp.dot(gpm, wsp_ref[...], preferred_element_type=jnp.float32)
          - jnp.dot(pt, wsq_ref[...], preferred_element_type=jnp.float32))
    sk = jnp.maximum(sk + bs_ref[...], 0.0)

    acc = jnp.dot(h[:, 0, :], wo1_ref[0], preferred_element_type=jnp.float32)
    for k in range(1, _K):
        acc = acc + jnp.dot(h[:, k, :], wo1_ref[k],
                            preferred_element_type=jnp.float32)
    out1 = jnp.maximum(acc + bo1_ref[...], 0.0) + sk
    out = jnp.maximum(jnp.dot(out1, wo2_ref[...], preferred_element_type=jnp.float32)
                      + bo2_ref[...], 0.0)
    o_ref[0] = out


def _mlp(g4, p8, weights):
    full = lambda shape: pl.BlockSpec(shape, lambda b, i: tuple(0 for _ in shape))
    w_specs = [
        full((128, 128)), full((8, 128)), full((8, 128)), full((1, 128)),
        full((128, 128)), full((1, 128)), full((128, 256)), full((1, 256)),
        full((128, 256)), full((8, 256)), full((8, 256)), full((1, 256)),
        full((_K, 256, 256)), full((1, 256)), full((256, 256)), full((1, 256)),
    ]
    return pl.pallas_call(
        _mlp_body,
        grid=(_B, _N // _TN),
        in_specs=[
            pl.BlockSpec((1, _TN, _K, _CP), lambda b, i: (b, i, 0, 0)),
            pl.BlockSpec((1, _TN, 8), lambda b, i: (b, i, 0)),
        ] + w_specs,
        out_specs=pl.BlockSpec((1, _TN, 256), lambda b, i: (b, i, 0)),
        out_shape=jax.ShapeDtypeStruct((_B, _N, 256), jnp.float32),
    )(g4, p8, *weights)


def kernel(points, point_features, query_points, W_skip, b_skip,
           W1, b1, W2, b2, W3, b3, Wo1, bo1, Wo2, bo2):
    f32 = jnp.float32
    pad5 = lambda x: jnp.pad(x, ((0, 0), (0, 0), (0, 5)))
    p8 = pad5(jnp.transpose(points, (0, 2, 1)))
    q8 = pad5(jnp.transpose(query_points, (0, 2, 1)))
    p8t = jnp.transpose(p8, (0, 2, 1))
    ft = jnp.transpose(point_features, (0, 2, 1))

    idx = _knn(q8, p8t)

    table = jnp.concatenate(
        [ft, p8[:, :, 0:3], jnp.zeros((_B, _N, _CP - _CIN - 3), f32)],
        axis=2).reshape(_B * _N, _CP)
    flat_idx = (idx + (jnp.arange(_B, dtype=jnp.int32) * _N)[:, None, None])
    flat_idx = flat_idx.reshape(1, _B * _N * _K)
    g = _sc_gather(table, flat_idx)
    g4 = g.reshape(_B, _N, _K, _CP)

    pad_w = lambda w: jnp.pad(w, ((0, 5), (0, 0)))
    W1t, W2t, W3t = W1.T, W2.T, W3.T
    Wst, Wo2t = W_skip.T, Wo2.T
    w1f = W1t[3:3 + _CIN, :]
    w1p = pad_w(W1t[0:3, :] + W1t[131:134, :])
    w1q = pad_w(W1t[131:134, :])
    wsf = Wst[3:3 + _CIN, :]
    wsp = pad_w(Wst[0:3, :] + Wst[131:134, :])
    wsq = pad_w(Wst[131:134, :])
    wo1t = jnp.transpose(Wo1, (1, 2, 0))
    row = lambda b: b.reshape(1, -1)
    weights = [w1f, w1p, w1q, row(b1), W2t, row(b2), W3t, row(b3),
               wsf, wsp, wsq, row(b_skip), wo1t, row(bo1), Wo2t, row(bo2)]

    h = _mlp(g4, p8, weights)
    return (points, jnp.transpose(h, (0, 2, 1)))
